# baseline (speedup 1.0000x reference)
import jax
import jax.numpy as jnp
from jax.experimental import pallas as pl

X_ID = 343
B_BLK = 64


def _build_kernel(tok_ref, vl_ref, prompts_ref, tps_ref, tv_ref, out_ref):
    t = tok_ref.shape[1]
    tok_row = tok_ref[...]
    xm_row = (tok_row == X_ID).astype(jnp.int32)
    r = jax.lax.broadcasted_iota(jnp.int32, (t, t), 0)
    c = jax.lax.broadcasted_iota(jnp.int32, (t, t), 1)
    cnt_incl = jnp.sum(jnp.where(c <= r, xm_row, 0), axis=1, keepdims=True)
    cnt_excl = jnp.sum(jnp.where(c < r, xm_row, 0), axis=1, keepdims=True)
    xm = (cnt_incl - cnt_excl) > 0
    cnt = cnt_incl - 1
    tmpl = tps_ref[0]
    for j in range(prompts_ref.shape[1]):
        tmpl = jnp.where(xm & (cnt == j), prompts_ref[0, j][None, :], tmpl)
    view_slot = xm & (cnt == prompts_ref.shape[1])
    labels = vl_ref[0]
    tv1 = tv_ref[0, 1][None, :]
    tv2 = tv_ref[0, 2][None, :]
    view_rows = jnp.where(labels == 0, tv1, tv2)
    out_ref[...] = jnp.where(
        view_slot[None, :, :], view_rows[:, None, :], tmpl[None, :, :]
    )


def kernel(view_label, prompts, token_prefix_suffix, token_view, tokenized_prompts):
    b = view_label.shape[0]
    t, d = token_prefix_suffix.shape[1], token_prefix_suffix.shape[2]
    n_blocks = b // B_BLK
    tok = tokenized_prompts.astype(jnp.int32).reshape(1, t)
    vl = view_label.astype(jnp.int32).reshape(n_blocks, B_BLK, 1)
    return pl.pallas_call(
        _build_kernel,
        grid=(n_blocks,),
        in_specs=[
            pl.BlockSpec((1, t), lambda i: (0, 0)),
            pl.BlockSpec((1, B_BLK, 1), lambda i: (i, 0, 0)),
            pl.BlockSpec((1, prompts.shape[1], d), lambda i: (0, 0, 0)),
            pl.BlockSpec((1, t, d), lambda i: (0, 0, 0)),
            pl.BlockSpec((1, t, d), lambda i: (0, 0, 0)),
        ],
        out_specs=pl.BlockSpec((B_BLK, t, d), lambda i: (i, 0, 0)),
        out_shape=jax.ShapeDtypeStruct((b, t, d), token_prefix_suffix.dtype),
    )(tok, vl, prompts, token_prefix_suffix, token_view)
